# baseline (speedup 1.0000x reference)
import functools

import jax
import jax.numpy as jnp
from jax import lax
from jax.experimental import pallas as pl
from jax.experimental.pallas import tpu as pltpu
from jax.experimental.pallas import tpu_sc as plsc

_B = 16384
_D = 64
_NC = 2
_NS = 16
_NW = _NC * _NS
_BPW = _B // _NW


@functools.cache
def _gather_kernel():
    mesh = plsc.VectorSubcoreMesh(
        core_axis_name="c", subcore_axis_name="s",
        num_cores=_NC, num_subcores=_NS,
    )

    @functools.partial(
        pl.kernel,
        out_type=jax.ShapeDtypeStruct((_B, _D), jnp.float32),
        mesh=mesh,
        scratch_types=[
            pltpu.VMEM((_BPW,), jnp.int32),
            pltpu.VMEM((_BPW, _D), jnp.float32),
            pltpu.SemaphoreType.DMA,
        ],
        compiler_params=pltpu.CompilerParams(use_tc_tiling_on_sc=False),
    )
    def body(idx_hbm, table_hbm, out_hbm, idx_v, rows_v, sem):
        wid = lax.axis_index("s") * _NC + lax.axis_index("c")
        base = wid * _BPW
        pltpu.sync_copy(idx_hbm.at[pl.ds(base, _BPW)], idx_v)
        pltpu.async_copy(table_hbm.at[idx_v], rows_v, sem).wait()
        pltpu.sync_copy(rows_v, out_hbm.at[pl.ds(base, _BPW)])

    return body


def kernel(class_label, table, uncond_table):
    del uncond_table
    idx = class_label.astype(jnp.int32)
    out = _gather_kernel()(idx, table)
    return out.reshape(_B, 1, _D)


# speedup vs baseline: 2.1529x; 2.1529x over previous
import functools

import jax
import jax.numpy as jnp
from jax import lax
from jax.experimental import pallas as pl
from jax.experimental.pallas import tpu as pltpu
from jax.experimental.pallas import tpu_sc as plsc

_B = 16384
_D = 64
_V = 1000000
_NC = 2
_NS = 16
_NW = _NC * _NS
_BPW = _B // _NW
_C = 16
_NCHUNK = _BPW // _C
_L = 16


@functools.cache
def _gather_kernel():
    mesh = plsc.VectorSubcoreMesh(
        core_axis_name="c", subcore_axis_name="s",
        num_cores=_NC, num_subcores=_NS,
    )

    @functools.partial(
        pl.kernel,
        out_type=jax.ShapeDtypeStruct((_B, _D), jnp.float32),
        mesh=mesh,
        scratch_types=[
            pltpu.VMEM((_BPW,), jnp.int32),
            pltpu.VMEM((_C, 8, _D), jnp.float32),
            pltpu.VMEM((_BPW, _D), jnp.float32),
            pltpu.SemaphoreType.DMA,
            pltpu.SemaphoreType.DMA,
        ],
    )
    def body(idx_hbm, table_hbm, out_hbm, lab_v, tiles_v,
             out_v, sem_in, sem_g):
        wid = lax.axis_index("s") * _NC + lax.axis_index("c")
        base = wid * _BPW
        pltpu.async_copy(idx_hbm.at[pl.ds(base, _BPW)], lab_v, sem_in).wait()

        def do_chunk(ch, _):
            off = ch * _C
            lab = lab_v[pl.ds(off, _C)]
            copies = []
            for e in range(_C):
                t = lax.shift_right_logical(lab[e], 3)
                copies.append(
                    pltpu.async_copy(table_hbm.at[t], tiles_v.at[e], sem_g)
                )
            for cp in copies:
                cp.wait()
            for e in range(_C):
                r = lax.bitwise_and(lab[e], 7)
                for c in range(_D // _L):
                    out_v[off + e, pl.ds(c * _L, _L)] = (
                        tiles_v[e, r, pl.ds(c * _L, _L)]
                    )
            return ()

        lax.fori_loop(0, _NCHUNK, do_chunk, ())
        pltpu.sync_copy(out_v, out_hbm.at[pl.ds(base, _BPW)])

    return body


def kernel(class_label, table, uncond_table):
    del uncond_table
    idx = class_label.astype(jnp.int32)
    table3 = table.reshape(_V // 8, 8, _D)
    out = _gather_kernel()(idx, table3)
    return out.reshape(_B, 1, _D)
